# baseline (speedup 1.0000x reference)
import functools

import jax
import jax.numpy as jnp
from jax import lax
from jax.experimental import pallas as pl
from jax.experimental.pallas import tpu as pltpu
from jax.experimental.pallas import tpu_sc as plsc

_MEM_SLOTS = 262144
_NW = 32
_CH = 512
_LCAP = 520


def _hash_kernel(x_ref, wslots_ref, qslots_ref):
    B, T, D = x_ref.shape
    x = x_ref[...]
    writes = x.reshape(B * T, D)
    s = jnp.sum(writes * 1000.0, axis=-1)
    h = jnp.floor(s).astype(jnp.int32)
    slots = jnp.mod(h, _MEM_SLOTS)[None, :]
    j = lax.broadcasted_iota(jnp.int32, (1, B * T), 1)
    valid = jnp.mod(j, T) != (T - 1)
    wslots_ref[...] = jnp.where(valid, slots, -1)

    q = x[:, T - 1, :]
    sq = jnp.sum(q * 1000.0, axis=-1)
    hq = jnp.floor(sq).astype(jnp.int32)
    qslots_ref[...] = jnp.mod(hq, _MEM_SLOTS)


def _sc_join(wslots_hbm, qslots_hbm, qexp_hbm, xflat_hbm, zin_hbm,
             partials_hbm,
             slots_v, qidx_v, qexp_v, mmbuf, wlist, slist, acc_v, rowbuf,
             sem):
    c = lax.axis_index("c")
    s = lax.axis_index("s")
    wid = s * 2 + c

    pltpu.sync_copy(zin_hbm, acc_v)
    pltpu.sync_copy(wslots_hbm.at[wid], slots_v)
    pltpu.sync_copy(qslots_hbm, qidx_v)
    pltpu.sync_copy(qexp_hbm, qexp_v)

    def cmp_body(k, mm):
        qb = qexp_v[pl.ds(k * 16, 16)]
        out = []
        for v in range(32):
            sv = slots_v[v // 8, pl.ds((v % 8) * 16, 16)]
            eqi = jnp.where(sv == qb, 1, 0)
            out.append(jnp.maximum(mm[v], eqi))
        return tuple(out)

    mm0 = tuple(jnp.zeros((16,), jnp.int32) for _ in range(32))
    mm = lax.fori_loop(0, 64, cmp_body, mm0)
    for v in range(32):
        mmbuf[pl.ds(v * 16, 16)] = mm[v]

    def gather_body(j, cnt):
        m16 = mmbuf[pl.ds(j * 16, 16)]
        s16 = slots_v[j // 8, pl.ds((j % 8) * 16, 16)]
        for l in range(16):
            matched = m16[l] > 0
            off = jnp.where(matched, cnt, _LCAP - 1)
            widx = wid * _CH + j * 16 + l
            wlist[pl.ds(off * 16, 16)] = jnp.full((16,), widx, jnp.int32)
            slist[pl.ds(off * 16, 16)] = jnp.full((16,), s16[l], jnp.int32)
            cnt = cnt + jnp.where(matched, 1, 0)
        return cnt

    cnt = lax.fori_loop(0, 32, gather_body, jnp.int32(0))

    def match_body(i, carry):
        widx = wlist[pl.ds(i * 16, 16)][0]
        sl = slist[pl.ds(i * 16, 16)][0]
        pltpu.sync_copy(xflat_hbm.at[pl.ds(widx, 1)], rowbuf)
        slv = jnp.full((16,), sl, jnp.int32)
        for g in range(4):
            qv16 = qidx_v[pl.ds(g * 16, 16)]
            meqi = jnp.where(qv16 == slv, 1, 0)
            for l2 in range(16):
                @pl.when(meqi[l2] > 0)
                def _add(g=g, l2=l2):
                    q = g * 16 + l2
                    for cc in range(8):
                        o = q * 128 + cc * 16
                        acc_v[pl.ds(o, 16)] = (
                            acc_v[pl.ds(o, 16)]
                            + rowbuf[0, pl.ds(cc * 16, 16)])
        return carry

    lax.fori_loop(0, cnt, match_body, jnp.int32(0))

    pltpu.sync_copy(acc_v, partials_hbm.at[wid])


def _finish_kernel(partials_ref, w_ref, b_ref, out_ref):
    NW, B, D = partials_ref.shape
    retrieved = jnp.sum(partials_ref[...], axis=0)
    out = lax.dot_general(
        retrieved, w_ref[...], (((1,), (1,)), ((), ())),
        precision=lax.Precision.HIGHEST,
        preferred_element_type=jnp.float32)
    out_ref[...] = out + b_ref[...][None, :]


def kernel(x, hx_list, W, b):
    del hx_list
    B, T, D = x.shape
    wslots, qslots = pl.pallas_call(
        _hash_kernel,
        out_shape=(jax.ShapeDtypeStruct((1, B * T), jnp.int32),
                   jax.ShapeDtypeStruct((B,), jnp.int32)),
    )(x)

    wslots3 = wslots.reshape(_NW, 4, 128)
    qexp = jnp.repeat(qslots, 16)
    xflat = x.reshape(B * T, D)
    zin = jnp.zeros((B * D,), jnp.float32)

    mesh = plsc.VectorSubcoreMesh(core_axis_name="c", subcore_axis_name="s")
    sc = pl.kernel(
        _sc_join, mesh=mesh,
        out_type=jax.ShapeDtypeStruct((_NW, B * D), jnp.float32),
        scratch_types=[
            pltpu.VMEM((4, 128), jnp.int32),
            pltpu.VMEM((B,), jnp.int32),
            pltpu.VMEM((B * 16,), jnp.int32),
            pltpu.VMEM((_CH,), jnp.int32),
            pltpu.VMEM((_LCAP * 16,), jnp.int32),
            pltpu.VMEM((_LCAP * 16,), jnp.int32),
            pltpu.VMEM((B * D,), jnp.float32),
            pltpu.VMEM((1, D), jnp.float32),
            pltpu.SemaphoreType.DMA,
        ],
    )
    partials = sc(wslots3, qslots, qexp, xflat, zin)
    partials = partials.reshape(_NW, B, D)

    return pl.pallas_call(
        _finish_kernel,
        out_shape=jax.ShapeDtypeStruct((B, D), x.dtype),
    )(partials, W, b)


# speedup vs baseline: 1.2319x; 1.2319x over previous
import jax
import jax.numpy as jnp
from jax import lax
from jax.experimental import pallas as pl
from jax.experimental.pallas import tpu as pltpu
from jax.experimental.pallas import tpu_sc as plsc

_MEM_SLOTS = 262144
_NW = 32
_CH = 512
_LCAP = 520


def _hash_kernel(x_ref, wslots_ref, anyf_ref, qslots_ref):
    B, T, D = x_ref.shape
    x = x_ref[...]
    writes = x.reshape(B * T, D)
    s = jnp.sum(writes * 1000.0, axis=-1)
    h = jnp.floor(s).astype(jnp.int32)
    slots = jnp.mod(h, _MEM_SLOTS)[None, :]
    j = lax.broadcasted_iota(jnp.int32, (1, B * T), 1)
    valid = jnp.mod(j, T) != (T - 1)
    wslots_ref[...] = jnp.where(valid, slots, -1)

    q = x[:, T - 1, :]
    sq = jnp.sum(q * 1000.0, axis=-1)
    hq = jnp.floor(sq).astype(jnp.int32)
    qs = jnp.mod(hq, _MEM_SLOTS)
    qslots_ref[...] = qs

    qcol = qs.astype(jnp.float32)[:, None]
    srow = slots.astype(jnp.float32)
    mask = jnp.where((qcol == srow) & valid, 1, 0)
    anyf_ref[...] = jnp.max(mask, axis=0)[None, :]


def _sc_join(wslots_hbm, anyf_hbm, qslots_hbm, xflat_hbm, zin_hbm,
             partials_hbm,
             slots_v, anyv, qidx_v, wlist, slist, acc_v, rowbuf, cntbuf,
             sem):
    c = lax.axis_index("c")
    s = lax.axis_index("s")
    wid = s * 2 + c

    pltpu.sync_copy(zin_hbm, acc_v)
    pltpu.sync_copy(wslots_hbm.at[wid], slots_v)
    pltpu.sync_copy(anyf_hbm.at[wid], anyv)
    pltpu.sync_copy(qslots_hbm, qidx_v)
    cntbuf[0] = jnp.int32(0)

    def scan_body(j, carry):
        m16 = anyv[j // 8, pl.ds((j % 8) * 16, 16)]
        s16 = slots_v[j // 8, pl.ds((j % 8) * 16, 16)]
        for l in range(16):
            @pl.when(m16[l] > 0)
            def _app(j=j, l=l, s16=s16):
                cnt = cntbuf[0]
                widx = wid * _CH + j * 16 + l
                wlist[pl.ds(cnt * 16, 16)] = jnp.full((16,), widx, jnp.int32)
                slist[pl.ds(cnt * 16, 16)] = jnp.full((16,), s16[l], jnp.int32)
                cntbuf[0] = cnt + 1
        return carry

    lax.fori_loop(0, 32, scan_body, jnp.int32(0))

    def match_body(i, carry):
        widx = wlist[pl.ds(i * 16, 16)][0]
        sl = slist[pl.ds(i * 16, 16)][0]
        pltpu.sync_copy(xflat_hbm.at[pl.ds(widx, 1)], rowbuf)
        slv = jnp.full((16,), sl, jnp.int32)
        for g in range(4):
            qv16 = qidx_v[pl.ds(g * 16, 16)]
            meqi = jnp.where(qv16 == slv, 1, 0)
            for l2 in range(16):
                @pl.when(meqi[l2] > 0)
                def _add(g=g, l2=l2):
                    qrow = g * 16 + l2
                    for cc in range(8):
                        o = qrow * 128 + cc * 16
                        acc_v[pl.ds(o, 16)] = (
                            acc_v[pl.ds(o, 16)]
                            + rowbuf[0, pl.ds(cc * 16, 16)])
        return carry

    lax.fori_loop(0, cntbuf[0], match_body, jnp.int32(0))

    pltpu.sync_copy(acc_v, partials_hbm.at[wid])


def _finish_kernel(partials_ref, w_ref, b_ref, out_ref):
    NW, B, D = partials_ref.shape
    retrieved = jnp.sum(partials_ref[...], axis=0)
    out = lax.dot_general(
        retrieved, w_ref[...], (((1,), (1,)), ((), ())),
        precision=lax.Precision.HIGHEST,
        preferred_element_type=jnp.float32)
    out_ref[...] = out + b_ref[...][None, :]


def kernel(x, hx_list, W, b):
    del hx_list
    B, T, D = x.shape
    wslots, anyf, qslots = pl.pallas_call(
        _hash_kernel,
        out_shape=(jax.ShapeDtypeStruct((1, B * T), jnp.int32),
                   jax.ShapeDtypeStruct((1, B * T), jnp.int32),
                   jax.ShapeDtypeStruct((B,), jnp.int32)),
    )(x)

    wslots3 = wslots.reshape(_NW, 4, 128)
    anyf3 = anyf.reshape(_NW, 4, 128)
    xflat = x.reshape(B * T, D)
    zin = jnp.zeros((B * D,), jnp.float32)

    mesh = plsc.VectorSubcoreMesh(core_axis_name="c", subcore_axis_name="s")
    sc = pl.kernel(
        _sc_join, mesh=mesh,
        out_type=jax.ShapeDtypeStruct((_NW, B * D), jnp.float32),
        scratch_types=[
            pltpu.VMEM((4, 128), jnp.int32),
            pltpu.VMEM((4, 128), jnp.int32),
            pltpu.VMEM((B,), jnp.int32),
            pltpu.VMEM((_LCAP * 16,), jnp.int32),
            pltpu.VMEM((_LCAP * 16,), jnp.int32),
            pltpu.VMEM((B * D,), jnp.float32),
            pltpu.VMEM((1, D), jnp.float32),
            pltpu.SMEM((1,), jnp.int32),
            pltpu.SemaphoreType.DMA,
        ],
    )
    partials = sc(wslots3, anyf3, qslots, xflat, zin)
    partials = partials.reshape(_NW, B, D)

    return pl.pallas_call(
        _finish_kernel,
        out_shape=jax.ShapeDtypeStruct((B, D), x.dtype),
    )(partials, W, b)
